# baseline (speedup 1.0000x reference)
import functools

import jax
import jax.numpy as jnp
from jax import lax
from jax.experimental import pallas as pl
from jax.experimental.pallas import tpu as pltpu
from jax.experimental.pallas import tpu_sc as plsc

_N = 10000
_E = 320000
_D = 128
_H = 128
_T = 3
_G = 64
_OUT = 10

_NC = 2
_NS = 16
_NW = _NC * _NS
_EPW = _E // _NW
_CH = 128
_NCH = 80
_PADE = _NCH * _CH - _EPW
_HCH = 40
_RPT = 632
_NPAD = _NS * _RPT

_BLK = 1000
_LN2 = 0.6931471805599453


def _ssp(v):
    return jnp.maximum(v, 0.0) + jnp.log1p(jnp.exp(-jnp.abs(v))) - _LN2


def _sc_body(q_hbm, srcp0_hbm, srcp1_hbm, dstp_hbm, zros_hbm, out_hbm,
             src_v, dst_v, bufa, bufb, dmy_v, agg_s, gsem):
    cc = lax.axis_index("c")
    s = lax.axis_index("s")
    w = cc * _NS + s
    pltpu.sync_copy(dstp_hbm.at[w], dst_v)
    pltpu.sync_copy(zros_hbm, bufa)
    for r in range(4):
        pltpu.sync_copy(bufa, agg_s.at[pl.ds(s * _RPT + r * _CH, _CH)])
    pltpu.sync_copy(bufa.at[pl.ds(0, _RPT - 4 * _CH)],
                    agg_s.at[pl.ds(s * _RPT + 4 * _CH, _RPT - 4 * _CH)])
    plsc.subcore_barrier()

    def one_chunk(buf, l, g):
        pltpu.async_copy(q_hbm.at[src_v.at[l]], buf, gsem).wait()
        pltpu.sync_copy(buf, agg_s.at[dst_v.at[g]], add=True)
        pltpu.sync_copy(zros_hbm.at[0], dmy_v)

    for half, srcp_hbm in enumerate((srcp0_hbm, srcp1_hbm)):
        pltpu.sync_copy(srcp_hbm.at[w], src_v)

        def pair(j, carry):
            one_chunk(bufa, 2 * j, half * _HCH + 2 * j)
            one_chunk(bufb, 2 * j + 1, half * _HCH + 2 * j + 1)
            return carry

        lax.fori_loop(0, _HCH // 2, pair, 0)

    plsc.subcore_barrier()
    pltpu.sync_copy(agg_s.at[pl.ds(s * _RPT, _RPT)],
                    out_hbm.at[cc].at[pl.ds(s * _RPT, _RPT)])


_sc_edge_agg = functools.partial(
    pl.kernel,
    mesh=plsc.VectorSubcoreMesh(core_axis_name="c", subcore_axis_name="s"),
    out_type=jax.ShapeDtypeStruct((_NC, _NPAD, _H), jnp.float32),
    scratch_types=[
        pltpu.VMEM((_HCH, _CH), jnp.int32),
        pltpu.VMEM((_NCH, _CH), jnp.int32),
        pltpu.VMEM((_CH, _H), jnp.float32),
        pltpu.VMEM((_CH, _H), jnp.float32),
        pltpu.VMEM((_CH,), jnp.float32),
        pltpu.VMEM_SHARED((_NPAD, _H), jnp.float32),
        pltpu.SemaphoreType.DMA,
    ],
)(_sc_body)


def _tc_first_body(x_ref, wi_ref, bi_ref, w1_ref, b1_ref, h_ref, q_ref):
    h = jnp.dot(x_ref[...], wi_ref[...],
                preferred_element_type=jnp.float32) + bi_ref[...]
    h_ref[...] = h
    q_ref[...] = _ssp(jnp.dot(h, w1_ref[...],
                              preferred_element_type=jnp.float32) + b1_ref[...])


_tc_first = pl.pallas_call(
    _tc_first_body,
    grid=(_N // _BLK,),
    in_specs=[
        pl.BlockSpec((_BLK, _D), lambda i: (i, 0)),
        pl.BlockSpec((_D, _H), lambda i: (0, 0)),
        pl.BlockSpec((1, _H), lambda i: (0, 0)),
        pl.BlockSpec((_H, _H), lambda i: (0, 0)),
        pl.BlockSpec((1, _H), lambda i: (0, 0)),
    ],
    out_specs=[pl.BlockSpec((_BLK, _H), lambda i: (i, 0)),
               pl.BlockSpec((_BLK, _H), lambda i: (i, 0))],
    out_shape=[jax.ShapeDtypeStruct((_N, _H), jnp.float32),
               jax.ShapeDtypeStruct((_N, _H), jnp.float32)],
)


def _tc_mid_body(h_ref, a_ref, w2_ref, b2_ref, w1_ref, b1_ref, ho_ref, q_ref):
    agg = a_ref[0] + a_ref[1]
    h = h_ref[...] + jnp.dot(agg, w2_ref[...],
                             preferred_element_type=jnp.float32) + b2_ref[...]
    ho_ref[...] = h
    q_ref[...] = _ssp(jnp.dot(h, w1_ref[...],
                              preferred_element_type=jnp.float32) + b1_ref[...])


_tc_mid = pl.pallas_call(
    _tc_mid_body,
    grid=(_N // _BLK,),
    in_specs=[
        pl.BlockSpec((_BLK, _H), lambda i: (i, 0)),
        pl.BlockSpec((_NC, _BLK, _H), lambda i: (0, i, 0)),
        pl.BlockSpec((_H, _H), lambda i: (0, 0)),
        pl.BlockSpec((1, _H), lambda i: (0, 0)),
        pl.BlockSpec((_H, _H), lambda i: (0, 0)),
        pl.BlockSpec((1, _H), lambda i: (0, 0)),
    ],
    out_specs=[pl.BlockSpec((_BLK, _H), lambda i: (i, 0)),
               pl.BlockSpec((_BLK, _H), lambda i: (i, 0))],
    out_shape=[jax.ShapeDtypeStruct((_N, _H), jnp.float32),
               jax.ShapeDtypeStruct((_N, _H), jnp.float32)],
)


def _tc_last_body(h_ref, a_ref, w2_ref, b2_ref, batch_ref,
                  wo1_ref, bo1_ref, wo2_ref, bo2_ref, out_ref, g_scr):
    i = pl.program_id(0)
    agg = a_ref[0] + a_ref[1]
    h = h_ref[...] + jnp.dot(agg, w2_ref[...],
                             preferred_element_type=jnp.float32) + b2_ref[...]
    onehot = (batch_ref[...] ==
              lax.broadcasted_iota(jnp.int32, (_BLK, _G), 1)).astype(jnp.float32)
    part = lax.dot_general(onehot, h, (((0,), (0,)), ((), ())),
                           preferred_element_type=jnp.float32)

    @pl.when(i == 0)
    def _():
        g_scr[...] = part

    @pl.when(i > 0)
    def _():
        g_scr[...] += part

    @pl.when(i == pl.num_programs(0) - 1)
    def _():
        g = g_scr[...]
        u = _ssp(jnp.dot(g, wo1_ref[...],
                         preferred_element_type=jnp.float32) + bo1_ref[...])
        out_ref[...] = jnp.dot(u, wo2_ref[...],
                               preferred_element_type=jnp.float32) + bo2_ref[...]


_tc_last = pl.pallas_call(
    _tc_last_body,
    grid=(_N // _BLK,),
    in_specs=[
        pl.BlockSpec((_BLK, _H), lambda i: (i, 0)),
        pl.BlockSpec((_NC, _BLK, _H), lambda i: (0, i, 0)),
        pl.BlockSpec((_H, _H), lambda i: (0, 0)),
        pl.BlockSpec((1, _H), lambda i: (0, 0)),
        pl.BlockSpec((_BLK, 1), lambda i: (i, 0)),
        pl.BlockSpec((_H, _H // 2), lambda i: (0, 0)),
        pl.BlockSpec((1, _H // 2), lambda i: (0, 0)),
        pl.BlockSpec((_H // 2, _OUT), lambda i: (0, 0)),
        pl.BlockSpec((1, _OUT), lambda i: (0, 0)),
    ],
    out_specs=pl.BlockSpec((_G, _OUT), lambda i: (0, 0)),
    out_shape=jax.ShapeDtypeStruct((_G, _OUT), jnp.float32),
    scratch_shapes=[pltpu.VMEM((_G, _H), jnp.float32)],
)


def kernel(x, edge_index, batch, W_in, b_in, W1, b1, W2, b2, Wo1, bo1, Wo2, bo2):
    src = edge_index[0].reshape(_NW, _EPW)
    dst = edge_index[1].reshape(_NW, _EPW)
    srcp = jnp.concatenate(
        [src, jnp.zeros((_NW, _PADE), jnp.int32)], axis=1).reshape(_NW, _NCH, _CH)
    srcp0 = srcp[:, :_HCH]
    srcp1 = srcp[:, _HCH:]
    dstp = jnp.concatenate(
        [dst, jnp.full((_NW, _PADE), _N, jnp.int32)], axis=1).reshape(_NW, _NCH, _CH)
    srcp = jnp.broadcast_to(
        (jnp.arange(_NW, dtype=jnp.int32) % _NS)[:, None, None] * _RPT
        + jnp.arange(_CH, dtype=jnp.int32)[None, None, :], (_NW, _NCH, _CH))
    zros = jnp.zeros((_CH, _H), jnp.float32)

    h, q = _tc_first(x, W_in, b_in.reshape(1, _H),
                     W1[0], b1[0].reshape(1, _H))
    agg = None
    for t in range(_T):
        agg = _sc_edge_agg(q, srcp0, srcp1, dstp, zros)
        if t < _T - 1:
            h, q = _tc_mid(h, agg, W2[t], b2[t].reshape(1, _H),
                           W1[t + 1], b1[t + 1].reshape(1, _H))
    out = _tc_last(h, agg, W2[_T - 1], b2[_T - 1].reshape(1, _H),
                   batch.reshape(_N, 1), Wo1, bo1.reshape(1, _H // 2),
                   Wo2, bo2.reshape(1, _OUT))
    return out


# speedup vs baseline: 1.0811x; 1.0811x over previous
import functools

import jax
import jax.numpy as jnp
from jax import lax
from jax.experimental import pallas as pl
from jax.experimental.pallas import tpu as pltpu
from jax.experimental.pallas import tpu_sc as plsc

_N = 10000
_E = 320000
_D = 128
_H = 128
_T = 3
_G = 64
_OUT = 10

_NC = 2
_NS = 16
_NW = _NC * _NS
_EPW = _E // _NW
_CH = 128
_NCH = 80
_PADE = _NCH * _CH - _EPW
_HCH = 40
_RPT = 632
_NPAD = _NS * _RPT

_BLK = 1000
_LN2 = 0.6931471805599453


def _ssp(v):
    return jnp.maximum(v, 0.0) + jnp.log1p(jnp.exp(-jnp.abs(v))) - _LN2


def _sc_body(q_hbm, srcp0_hbm, srcp1_hbm, dstp_hbm, zros_hbm, out_hbm,
             src_v, dst_v, bufa, bufb, dmy_v, agg_s, gsem):
    cc = lax.axis_index("c")
    s = lax.axis_index("s")
    w = cc * _NS + s
    pltpu.sync_copy(dstp_hbm.at[w], dst_v)
    pltpu.sync_copy(zros_hbm, bufa)
    for r in range(4):
        pltpu.sync_copy(bufa, agg_s.at[pl.ds(s * _RPT + r * _CH, _CH)])
    pltpu.sync_copy(bufa.at[pl.ds(0, _RPT - 4 * _CH)],
                    agg_s.at[pl.ds(s * _RPT + 4 * _CH, _RPT - 4 * _CH)])
    plsc.subcore_barrier()

    def one_chunk(buf, l, g):
        pltpu.async_copy(q_hbm.at[src_v.at[l]], buf, gsem).wait()
        pltpu.sync_copy(buf, agg_s.at[dst_v.at[g]], add=True)

    for half, srcp_hbm in enumerate((srcp0_hbm, srcp1_hbm)):
        pltpu.sync_copy(srcp_hbm.at[w], src_v)

        def pair(j, carry):
            one_chunk(bufa, 2 * j, half * _HCH + 2 * j)
            one_chunk(bufb, 2 * j + 1, half * _HCH + 2 * j + 1)
            return carry

        lax.fori_loop(0, _HCH // 2, pair, 0)

    plsc.subcore_barrier()
    pltpu.sync_copy(agg_s.at[pl.ds(s * _RPT, _RPT)],
                    out_hbm.at[cc].at[pl.ds(s * _RPT, _RPT)])


_sc_edge_agg = functools.partial(
    pl.kernel,
    mesh=plsc.VectorSubcoreMesh(core_axis_name="c", subcore_axis_name="s"),
    out_type=jax.ShapeDtypeStruct((_NC, _NPAD, _H), jnp.float32),
    scratch_types=[
        pltpu.VMEM((_HCH, _CH), jnp.int32),
        pltpu.VMEM((_NCH, _CH), jnp.int32),
        pltpu.VMEM((_CH, _H), jnp.float32),
        pltpu.VMEM((_CH, _H), jnp.float32),
        pltpu.VMEM((_CH,), jnp.float32),
        pltpu.VMEM_SHARED((_NPAD, _H), jnp.float32),
        pltpu.SemaphoreType.DMA,
    ],
)(_sc_body)


def _tc_first_body(x_ref, wi_ref, bi_ref, w1_ref, b1_ref, h_ref, q_ref):
    h = jnp.dot(x_ref[...], wi_ref[...],
                preferred_element_type=jnp.float32) + bi_ref[...]
    h_ref[...] = h
    q_ref[...] = _ssp(jnp.dot(h, w1_ref[...],
                              preferred_element_type=jnp.float32) + b1_ref[...])


_tc_first = pl.pallas_call(
    _tc_first_body,
    grid=(_N // _BLK,),
    in_specs=[
        pl.BlockSpec((_BLK, _D), lambda i: (i, 0)),
        pl.BlockSpec((_D, _H), lambda i: (0, 0)),
        pl.BlockSpec((1, _H), lambda i: (0, 0)),
        pl.BlockSpec((_H, _H), lambda i: (0, 0)),
        pl.BlockSpec((1, _H), lambda i: (0, 0)),
    ],
    out_specs=[pl.BlockSpec((_BLK, _H), lambda i: (i, 0)),
               pl.BlockSpec((_BLK, _H), lambda i: (i, 0))],
    out_shape=[jax.ShapeDtypeStruct((_N, _H), jnp.float32),
               jax.ShapeDtypeStruct((_N, _H), jnp.float32)],
)


def _tc_mid_body(h_ref, a_ref, w2_ref, b2_ref, w1_ref, b1_ref, ho_ref, q_ref):
    agg = a_ref[0] + a_ref[1]
    h = h_ref[...] + jnp.dot(agg, w2_ref[...],
                             preferred_element_type=jnp.float32) + b2_ref[...]
    ho_ref[...] = h
    q_ref[...] = _ssp(jnp.dot(h, w1_ref[...],
                              preferred_element_type=jnp.float32) + b1_ref[...])


_tc_mid = pl.pallas_call(
    _tc_mid_body,
    grid=(_N // _BLK,),
    in_specs=[
        pl.BlockSpec((_BLK, _H), lambda i: (i, 0)),
        pl.BlockSpec((_NC, _BLK, _H), lambda i: (0, i, 0)),
        pl.BlockSpec((_H, _H), lambda i: (0, 0)),
        pl.BlockSpec((1, _H), lambda i: (0, 0)),
        pl.BlockSpec((_H, _H), lambda i: (0, 0)),
        pl.BlockSpec((1, _H), lambda i: (0, 0)),
    ],
    out_specs=[pl.BlockSpec((_BLK, _H), lambda i: (i, 0)),
               pl.BlockSpec((_BLK, _H), lambda i: (i, 0))],
    out_shape=[jax.ShapeDtypeStruct((_N, _H), jnp.float32),
               jax.ShapeDtypeStruct((_N, _H), jnp.float32)],
)


def _tc_last_body(h_ref, a_ref, w2_ref, b2_ref, batch_ref,
                  wo1_ref, bo1_ref, wo2_ref, bo2_ref, out_ref, g_scr):
    i = pl.program_id(0)
    agg = a_ref[0] + a_ref[1]
    h = h_ref[...] + jnp.dot(agg, w2_ref[...],
                             preferred_element_type=jnp.float32) + b2_ref[...]
    onehot = (batch_ref[...] ==
              lax.broadcasted_iota(jnp.int32, (_BLK, _G), 1)).astype(jnp.float32)
    part = lax.dot_general(onehot, h, (((0,), (0,)), ((), ())),
                           preferred_element_type=jnp.float32)

    @pl.when(i == 0)
    def _():
        g_scr[...] = part

    @pl.when(i > 0)
    def _():
        g_scr[...] += part

    @pl.when(i == pl.num_programs(0) - 1)
    def _():
        g = g_scr[...]
        u = _ssp(jnp.dot(g, wo1_ref[...],
                         preferred_element_type=jnp.float32) + bo1_ref[...])
        out_ref[...] = jnp.dot(u, wo2_ref[...],
                               preferred_element_type=jnp.float32) + bo2_ref[...]


_tc_last = pl.pallas_call(
    _tc_last_body,
    grid=(_N // _BLK,),
    in_specs=[
        pl.BlockSpec((_BLK, _H), lambda i: (i, 0)),
        pl.BlockSpec((_NC, _BLK, _H), lambda i: (0, i, 0)),
        pl.BlockSpec((_H, _H), lambda i: (0, 0)),
        pl.BlockSpec((1, _H), lambda i: (0, 0)),
        pl.BlockSpec((_BLK, 1), lambda i: (i, 0)),
        pl.BlockSpec((_H, _H // 2), lambda i: (0, 0)),
        pl.BlockSpec((1, _H // 2), lambda i: (0, 0)),
        pl.BlockSpec((_H // 2, _OUT), lambda i: (0, 0)),
        pl.BlockSpec((1, _OUT), lambda i: (0, 0)),
    ],
    out_specs=pl.BlockSpec((_G, _OUT), lambda i: (0, 0)),
    out_shape=jax.ShapeDtypeStruct((_G, _OUT), jnp.float32),
    scratch_shapes=[pltpu.VMEM((_G, _H), jnp.float32)],
)


def kernel(x, edge_index, batch, W_in, b_in, W1, b1, W2, b2, Wo1, bo1, Wo2, bo2):
    src = edge_index[0].reshape(_NW, _EPW)
    dst = edge_index[1].reshape(_NW, _EPW)
    srcp = jnp.concatenate(
        [src, jnp.zeros((_NW, _PADE), jnp.int32)], axis=1).reshape(_NW, _NCH, _CH)
    srcp0 = srcp[:, :_HCH]
    srcp1 = srcp[:, _HCH:]
    dstp = jnp.concatenate(
        [dst, jnp.full((_NW, _PADE), _N, jnp.int32)], axis=1).reshape(_NW, _NCH, _CH)
    srcp = jnp.broadcast_to(
        (jnp.arange(_NW, dtype=jnp.int32) % _NS)[:, None, None] * _RPT
        + jnp.arange(_CH, dtype=jnp.int32)[None, None, :], (_NW, _NCH, _CH))
    zros = jnp.zeros((_CH, _H), jnp.float32)

    h, q = _tc_first(x, W_in, b_in.reshape(1, _H),
                     W1[0], b1[0].reshape(1, _H))
    agg = None
    for t in range(_T):
        agg = _sc_edge_agg(q, srcp0, srcp1, dstp, zros)
        if t < _T - 1:
            h, q = _tc_mid(h, agg, W2[t], b2[t].reshape(1, _H),
                           W1[t + 1], b1[t + 1].reshape(1, _H))
    out = _tc_last(h, agg, W2[_T - 1], b2[_T - 1].reshape(1, _H),
                   batch.reshape(_N, 1), Wo1, bo1.reshape(1, _H // 2),
                   Wo2, bo2.reshape(1, _OUT))
    return out
